# Initial kernel scaffold; baseline (speedup 1.0000x reference)
import jax
import jax.numpy as jnp
from jax.experimental import pallas as pl


def kernel(x, Wg, W1, b1, W2, b2, gamma, beta):
    raise NotImplementedError("write your pallas kernel here")


# speedup vs baseline: 1.6293x; 1.6293x over previous
import jax
import jax.numpy as jnp
from jax.experimental import pallas as pl

E = 8
TBLK = 256


def _moe_body(x_ref, wg_ref, w1_ref, b1_ref, w2_ref, b2_ref, g_ref, be_ref,
              out_ref):
    e = pl.program_id(0)
    tb = pl.program_id(1)
    xb = x_ref[...]

    logits = jax.lax.dot_general(
        xb, wg_ref[...], (((1,), (1,)), ((), ())),
        preferred_element_type=jnp.float32)
    eidx = jax.lax.broadcasted_iota(jnp.int32, logits.shape, 1)
    m1 = jnp.max(logits, axis=1, keepdims=True)
    i1 = jnp.min(jnp.where(logits == m1, eidx, E), axis=1, keepdims=True)
    l2 = jnp.where(eidx == i1, -jnp.inf, logits)
    m2 = jnp.max(l2, axis=1, keepdims=True)
    i2 = jnp.min(jnp.where(l2 == m2, eidx, E), axis=1, keepdims=True)
    p = jnp.exp(m2 - m1)
    w_top1 = 1.0 / (1.0 + p)
    w_top2 = p / (1.0 + p)
    we = jnp.where(i1 == e, w_top1, jnp.where(i2 == e, w_top2, 0.0))

    h = jnp.maximum(
        jnp.dot(xb, w1_ref[0], preferred_element_type=jnp.float32)
        + b1_ref[0], 0.0)
    y = jnp.dot(h, w2_ref[0], preferred_element_type=jnp.float32) + b2_ref[0]
    mu = jnp.mean(y, axis=1, keepdims=True)
    yc = y - mu
    var = jnp.mean(yc * yc, axis=1, keepdims=True)
    yn = yc * jax.lax.rsqrt(var + 1e-5) * g_ref[0] + be_ref[0]
    contrib = we * yn

    sl = pl.ds(tb * TBLK, TBLK)

    @pl.when(e == 0)
    def _():
        out_ref[sl, :] = contrib

    @pl.when(e != 0)
    def _():
        out_ref[sl, :] += contrib


def kernel(x, Wg, W1, b1, W2, b2, gamma, beta, interpret=False):
    Bsz, Slen, D = x.shape
    T = Bsz * Slen
    H = W1.shape[2]
    xf = x.reshape(T, D)
    nt = T // TBLK

    out = pl.pallas_call(
        _moe_body,
        grid=(E, nt),
        in_specs=[
            pl.BlockSpec((TBLK, D), lambda e, tb: (tb, 0)),
            pl.BlockSpec((E, D), lambda e, tb: (0, 0)),
            pl.BlockSpec((1, D, H), lambda e, tb: (e, 0, 0)),
            pl.BlockSpec((1, 1, H), lambda e, tb: (e, 0, 0)),
            pl.BlockSpec((1, H, D), lambda e, tb: (e, 0, 0)),
            pl.BlockSpec((1, 1, D), lambda e, tb: (e, 0, 0)),
            pl.BlockSpec((1, 1, D), lambda e, tb: (e, 0, 0)),
            pl.BlockSpec((1, 1, D), lambda e, tb: (e, 0, 0)),
        ],
        out_specs=pl.BlockSpec((T, D), lambda e, tb: (0, 0)),
        out_shape=jax.ShapeDtypeStruct((T, D), jnp.float32),
        interpret=interpret,
    )(xf, Wg, W1, b1.reshape(E, 1, H), W2, b2.reshape(E, 1, D),
      gamma.reshape(E, 1, D), beta.reshape(E, 1, D))
    return out.reshape(Bsz, Slen, D)
